# baseline (speedup 1.0000x reference)
import functools

import jax
import jax.numpy as jnp
from jax import lax
from jax.experimental import pallas as pl
from jax.experimental.pallas import tpu as pltpu

BM = 512


def _tc_body(ids_ref, x_ref, table_ref, out_ref):
    i = pl.program_id(0)
    base = i * BM

    def row4(r4, _):
        for u in range(4):
            r = r4 * 4 + u
            idx = ids_ref[base + r]
            out_ref[r] = x_ref[r] + table_ref[idx]
        return 0

    lax.fori_loop(0, BM // 4, row4, 0)


def _make_tc(n_rows, hidden):
    sub = hidden // 128
    grid_spec = pltpu.PrefetchScalarGridSpec(
        num_scalar_prefetch=1,
        grid=(n_rows // BM,),
        in_specs=[
            pl.BlockSpec((BM, sub, 128), lambda i, ids: (i, 0, 0)),
            pl.BlockSpec((2048, sub, 128), lambda i, ids: (0, 0, 0)),
        ],
        out_specs=pl.BlockSpec((BM, sub, 128), lambda i, ids: (i, 0, 0)),
    )
    return pl.pallas_call(
        _tc_body,
        grid_spec=grid_spec,
        out_shape=jax.ShapeDtypeStruct((n_rows, sub, 128), jnp.float32),
    )


@jax.jit
def kernel(X, position_ids, table):
    b, s, h = X.shape
    n = b * s
    sub = h // 128
    x3 = X.reshape(n, sub, 128)
    t3 = table.reshape(table.shape[0], sub, 128)
    ids = position_ids.reshape(n).astype(jnp.int32)
    out = _make_tc(n, h)(ids, x3, t3)
    return out.reshape(b, s, h)


# speedup vs baseline: 1.5670x; 1.5670x over previous
import functools

import jax
import jax.numpy as jnp
from jax import lax
from jax.experimental import pallas as pl
from jax.experimental.pallas import tpu as pltpu
from jax.experimental.pallas import tpu_sc as plsc

NUM_CORES = 2
NUM_SUBCORES = 16
NUM_WORKERS = NUM_CORES * NUM_SUBCORES
LANES = 16
NBUF = 3
CHUNK = 16
BM = 512
SC_ROWS = 4096


def _make_sc(n_rows, sc_rows, hidden):
    rows_per_w = sc_rows // NUM_WORKERS
    n_chunks = rows_per_w // CHUNK
    vecs_per_row = hidden // LANES
    assert sc_rows % (NUM_WORKERS * CHUNK) == 0 and n_chunks > NBUF
    mesh = plsc.VectorSubcoreMesh(core_axis_name="c", subcore_axis_name="s")

    @functools.partial(
        pl.kernel,
        mesh=mesh,
        out_type=jax.ShapeDtypeStruct((sc_rows, hidden), jnp.float32),
        scratch_types=(
            [pltpu.VMEM((rows_per_w,), jnp.int32)]
            + [pltpu.VMEM((CHUNK, hidden), jnp.float32) for _ in range(2 * NBUF)]
            + [pltpu.SemaphoreType.DMA for _ in range(3 * NBUF)]
        ),
    )
    def k(x_hbm, ids_hbm, table_hbm, out_hbm, idx_all, *rest):
        xbufs = rest[0:NBUF]
        rbufs = rest[NBUF:2 * NBUF]
        sem_x = rest[2 * NBUF:3 * NBUF]
        sem_r = rest[3 * NBUF:4 * NBUF]
        sem_o = rest[4 * NBUF:5 * NBUF]

        wid = lax.axis_index("s") * NUM_CORES + lax.axis_index("c")
        base0 = wid * rows_per_w
        pltpu.sync_copy(ids_hbm.at[pl.ds(base0, rows_per_w)], idx_all)

        in_flight = {}
        out_flight = {}

        def start_in(t):
            b = t % NBUF
            cx = pltpu.async_copy(
                x_hbm.at[pl.ds(base0 + t * CHUNK, CHUNK)], xbufs[b], sem_x[b])
            cr = pltpu.async_copy(
                table_hbm.at[idx_all.at[pl.ds(t * CHUNK, CHUNK)]],
                rbufs[b], sem_r[b])
            in_flight[t] = (cx, cr)

        def compute(b):
            xb, rb = xbufs[b], rbufs[b]

            def add_row(r, _):
                for j in range(vecs_per_row):
                    plsc.addupdate(
                        xb.at[r, pl.ds(j * LANES, LANES)],
                        rb[r, pl.ds(j * LANES, LANES)])
                return 0

            lax.fori_loop(0, CHUNK, add_row, 0)

        start_in(0)
        for t in range(n_chunks):
            if t + 1 < n_chunks:
                if t + 1 >= NBUF:
                    out_flight.pop(t + 1 - NBUF).wait()
                start_in(t + 1)
            b = t % NBUF
            cx, cr = in_flight.pop(t)
            cx.wait()
            cr.wait()
            compute(b)
            out_flight[t] = pltpu.async_copy(
                xbufs[b], out_hbm.at[pl.ds(base0 + t * CHUNK, CHUNK)], sem_o[b])
        for t in sorted(out_flight):
            out_flight.pop(t).wait()

    return k


def _tc_body(ids_ref, x_ref, table_ref, out_ref):
    ids_col = ids_ref[...]
    cols = lax.broadcasted_iota(jnp.int32, (1, table_ref.shape[0]), 1)
    onehot = (ids_col == cols).astype(jnp.bfloat16)
    pe = jnp.dot(onehot, table_ref[...], preferred_element_type=jnp.float32)
    out_ref[...] = x_ref[...] + pe


def _make_tc(n_rows, sc_rows, hidden, vocab):
    tc_rows = n_rows - sc_rows
    assert tc_rows % BM == 0
    off = sc_rows // BM
    return pl.pallas_call(
        _tc_body,
        grid=(tc_rows // BM,),
        in_specs=[
            pl.BlockSpec((BM, 1), lambda i: (i + off, 0)),
            pl.BlockSpec((BM, hidden), lambda i: (i + off, 0)),
            pl.BlockSpec((vocab, hidden), lambda i: (0, 0)),
        ],
        out_specs=pl.BlockSpec((BM, hidden), lambda i: (i, 0)),
        out_shape=jax.ShapeDtypeStruct((tc_rows, hidden), jnp.float32),
    )


@jax.jit
def kernel(X, position_ids, table):
    b, s, h = X.shape
    n = b * s
    v = table.shape[0]
    x2d = X.reshape(n, h)
    ids = position_ids.reshape(n).astype(jnp.int32)
    out_sc = _make_sc(n, SC_ROWS, h)(x2d, ids, table)
    table_bf = table.astype(jnp.bfloat16)
    out_tc = _make_tc(n, SC_ROWS, h, v)(ids.reshape(n, 1), x2d, table_bf)
    out = jnp.concatenate([out_sc, out_tc], axis=0)
    return out.reshape(b, s, h)
